# baseline (speedup 1.0000x reference)
import functools

import jax
import jax.numpy as jnp
from jax import lax
from jax.experimental import pallas as pl
from jax.experimental.pallas import tpu as pltpu
from jax.experimental.pallas import tpu_sc as plsc

N = 10000
E = 320000
D = 128
NUM_ETYPES = 8
NUM_WKR = 5000

NC = 2
NS = 16
NW = NC * NS
PER_W = E // NW
CHUNK = 128
NROWS = PER_W // CHUNK + 4

M = 5000
H = 5008
DUMP = 5000

NODE_B = 2000

_MESH = plsc.VectorSubcoreMesh(core_axis_name="c", subcore_axis_name="s")


def _tbuild_body(h_ref, e2d_ref, w1_ref, b1_ref, t_ref):
    w1a = w1_ref[pl.ds(0, D), :]
    w1b = w1_ref[pl.ds(D, 16), :]
    hh = jnp.dot(h_ref[...], w1a, preferred_element_type=jnp.float32)
    c8 = jnp.dot(e2d_ref[...], w1b,
                 preferred_element_type=jnp.float32) + b1_ref[...]
    t_ref[...] = hh[:, None, :] + c8[None, :, :]


def _tc_tbuild(h, e2d, w1, b1):
    return pl.pallas_call(
        _tbuild_body,
        grid=(N // NODE_B,),
        in_specs=[
            pl.BlockSpec((NODE_B, D), lambda i: (i, 0)),
            pl.BlockSpec((NUM_ETYPES, 16), lambda i: (0, 0)),
            pl.BlockSpec((D + 16, D), lambda i: (0, 0)),
            pl.BlockSpec((1, D), lambda i: (0, 0)),
        ],
        out_specs=pl.BlockSpec((NODE_B, NUM_ETYPES, D), lambda i: (i, 0, 0)),
        out_shape=jax.ShapeDtypeStruct((N, NUM_ETYPES, D), jnp.float32),
    )(h, e2d, w1, b1)


def _make_sc_edge(lo):
    hi = lo + M

    @functools.partial(
        pl.kernel,
        out_type=(
            jax.ShapeDtypeStruct((NC * H, D), jnp.float32),
            jax.ShapeDtypeStruct((NW * H,), jnp.float32),
        ),
        mesh=_MESH,
        scratch_types=[
            pltpu.VMEM((PER_W + 2 * CHUNK + 16,), jnp.int32),
            pltpu.VMEM((PER_W + 2 * CHUNK + 16,), jnp.int32),
            pltpu.VMEM((NROWS, CHUNK), jnp.int32),
            pltpu.VMEM((D,), jnp.float32),
            pltpu.VMEM((CHUNK, D), jnp.float32),
            pltpu.VMEM((CHUNK, D), jnp.float32),
            pltpu.VMEM((H,), jnp.float32),
            pltpu.VMEM_SHARED((H, D), jnp.float32),
            pltpu.SemaphoreType.DMA,
            pltpu.SemaphoreType.DMA,
        ],
        compiler_params=pltpu.CompilerParams(needs_layout_passes=False),
    )
    def sc_edge(t_hbm, src_hbm, et_hbm, dst_hbm, w3_hbm, zrow_hbm, zden_hbm,
                num_hbm, den_hbm, idx8, stg, didx2, w3v, rows_a,
                rows_b, den_l, sh_num, sem_a, sem_b):
        cid = lax.axis_index("c")
        sid = lax.axis_index("s")
        wid = sid * NC + cid
        base = wid * PER_W

        @pl.when(sid == 0)
        def _():
            pltpu.sync_copy(zrow_hbm, sh_num)

        pltpu.sync_copy(zden_hbm, den_l)
        pltpu.sync_copy(w3_hbm, w3v)
        pltpu.sync_copy(src_hbm.at[pl.ds(base, PER_W)], idx8.at[pl.ds(0, PER_W)])
        pltpu.sync_copy(et_hbm.at[pl.ds(base, PER_W)], stg.at[pl.ds(0, PER_W)])

        def mk_idx(g, carry):
            off = g * 16
            idx8[pl.ds(off, 16)] = idx8[pl.ds(off, 16)] * 8 + stg[pl.ds(off, 16)]
            return carry

        lax.fori_loop(0, PER_W // 16, mk_idx, 0)
        pltpu.sync_copy(dst_hbm.at[pl.ds(base, PER_W)], stg.at[pl.ds(0, PER_W)])

        def compact(g, cnt):
            off = g * 16
            dv = stg[pl.ds(off, 16)]
            iv = idx8[pl.ds(off, 16)]
            inr = (dv >= lo) & (dv < hi)
            plsc.store_compressed(idx8.at[pl.ds(cnt, 16)], iv, mask=inr)
            plsc.store_compressed(stg.at[pl.ds(cnt, 16)], dv - lo, mask=inr)
            return cnt + jnp.sum(inr.astype(jnp.int32))

        n_w = lax.fori_loop(0, PER_W // 16, compact, jnp.int32(0))
        for jj in range(2 * CHUNK // 16):
            idx8[pl.ds(n_w + jj * 16, 16)] = jnp.zeros((16,), jnp.int32)
            stg[pl.ds(n_w + jj * 16, 16)] = jnp.full((16,), DUMP, jnp.int32)
        nci = (n_w + CHUNK - 1) // CHUNK
        nci = nci + 1 - (nci & 1)

        def mk_rows(g2, carry):
            didx2[g2 // (CHUNK // 16),
                  pl.ds((g2 % (CHUNK // 16)) * 16, 16)] = stg[pl.ds(g2 * 16, 16)]
            return carry

        lax.fori_loop(0, nci * (CHUNK // 16), mk_rows, 0)
        plsc.subcore_barrier()

        lane0 = lax.iota(jnp.int32, 16) == 0

        def start(ci, buf, sem):
            pltpu.async_copy(
                t_hbm.at[idx8.at[pl.ds(ci * CHUNK, CHUNK)]], buf, sem)

        def finish(ci, buf, sem):
            pltpu.make_async_copy(
                t_hbm.at[idx8.at[pl.ds(ci * CHUNK, CHUNK)]], buf, sem).wait()

            @plsc.parallel_loop(0, CHUNK, step=1, unroll=4)
            def edge_body(e):
                sigs = []
                acc = None
                for j in range(D // 16):
                    t = buf[e, pl.ds(j * 16, 16)]
                    sj = 1.0 / (1.0 + jnp.exp(-t))
                    sigs.append(sj)
                    term = sj * w3v[pl.ds(j * 16, 16)]
                    acc = term if acc is None else acc + term
                s = jnp.sum(acc)
                exv = jnp.exp(jnp.full((16,), s, jnp.float32))
                for j in range(D // 16):
                    buf[e, pl.ds(j * 16, 16)] = sigs[j] * exv
                dmsplat = plsc.load_gather(
                    stg, [jnp.full((16,), ci * CHUNK + e, jnp.int32)])
                plsc.addupdate_scatter(den_l, [dmsplat], exv, mask=lane0)
            pltpu.sync_copy(buf, sh_num.at[didx2.at[ci]], add=True)

        start(0, rows_a, sem_a)

        def body(k, carry):
            start(2 * k + 1, rows_b, sem_b)
            finish(2 * k, rows_a, sem_a)
            start(2 * k + 2, rows_a, sem_a)
            finish(2 * k + 1, rows_b, sem_b)
            return carry

        lax.fori_loop(0, (nci - 1) // 2, body, 0)
        finish(nci - 1, rows_a, sem_a)
        plsc.subcore_barrier()
        pltpu.sync_copy(den_l, den_hbm.at[pl.ds(wid * H, H)])

        @pl.when(sid == 0)
        def _():
            pltpu.sync_copy(sh_num, num_hbm.at[pl.ds(cid * H, H)])

    return sc_edge


_SC_EDGE = (_make_sc_edge(0), _make_sc_edge(M))


def _update_body(num_ref, den_ref, h_ref, w2_ref, b2_ref, out_ref,
                 *, alpha, beta, lo):
    ones = jnp.ones((NW, 1), jnp.float32)
    den = lax.dot_general(den_ref[...], ones, (((0,), (0,)), ((), ())),
                          preferred_element_type=jnp.float32)
    num = num_ref[pl.ds(0, H), :] + num_ref[pl.ds(H, H), :]
    agg = (num / jnp.maximum(den, 1e-30))[:M, :]
    z = alpha * agg + beta * h_ref[pl.ds(lo, M), :]
    z = jax.nn.sigmoid(jnp.dot(z, w2_ref[...],
                               preferred_element_type=jnp.float32) + b2_ref[...])
    if lo == 0:
        out_ref[pl.ds(0, M), :] = z
        out_ref[pl.ds(M, N - M), :] = h_ref[pl.ds(M, N - M), :]
    else:
        out_ref[pl.ds(0, lo), :] = h_ref[pl.ds(0, lo), :]
        out_ref[pl.ds(lo, M), :] = z


def _tc_update(num, den, h, w2, b2, alpha, beta, lo):
    body = functools.partial(_update_body, alpha=alpha, beta=beta, lo=lo)
    return pl.pallas_call(
        body,
        out_shape=jax.ShapeDtypeStruct((N, D), jnp.float32),
    )(num, den, h, w2, b2)


def kernel(features, edge_index, edge_type, w1_wkr, b1_wkr, w1_tsk, b1_tsk,
           w2_wkr, b2_wkr, w2_tsk, b2_tsk, w3_wkr, b3_wkr, w3_tsk, b3_tsk,
           e_wkr, e_tsk):
    h0 = features[:, 0, :]
    src = edge_index[0].astype(jnp.int32)
    dst = edge_index[1].astype(jnp.int32)
    et = edge_type.astype(jnp.int32)
    zrow = jnp.zeros((H, D), jnp.float32)
    zden = jnp.zeros((H,), jnp.float32)

    def stage(h, e_tab, w1, b1, w3, idx):
        t_tab = _tc_tbuild(h, e_tab[:, 0, :], w1, b1).reshape(N * NUM_ETYPES, D)
        numf, denf = _SC_EDGE[idx](t_tab, src, et, dst, w3[:D, 0], zrow, zden)
        return numf, denf.reshape(NW, H)

    num, den = stage(h0, e_wkr, w1_wkr, b1_wkr, w3_wkr, 0)
    h1 = _tc_update(num, den, h0, w2_wkr, b2_wkr, 1.0, 0.5, 0)
    num, den = stage(h1, e_tsk, w1_tsk, b1_tsk, w3_tsk, 1)
    h2 = _tc_update(num, den, h1, w2_tsk, b2_tsk, 0.7, 0.3, NUM_WKR)
    return h2[:, None, :]


# speedup vs baseline: 1.1726x; 1.1726x over previous
import functools

import jax
import jax.numpy as jnp
from jax import lax
from jax.experimental import pallas as pl
from jax.experimental.pallas import tpu as pltpu
from jax.experimental.pallas import tpu_sc as plsc

N = 10000
E = 320000
D = 128
NUM_ETYPES = 8
NUM_WKR = 5000

NC = 2
NS = 16
NW = NC * NS
PER_W = E // NW
CHUNK = 80
NROWS = PER_W // CHUNK + 4

M = 5000
H = 5008
DUMP = 5000

NODE_B = 2000

_MESH = plsc.VectorSubcoreMesh(core_axis_name="c", subcore_axis_name="s")


def _tbuild_body(h_ref, e2d_ref, w1_ref, b1_ref, t_ref):
    w1a = w1_ref[pl.ds(0, D), :]
    w1b = w1_ref[pl.ds(D, 16), :]
    hh = jnp.dot(h_ref[...], w1a, preferred_element_type=jnp.float32)
    c8 = jnp.dot(e2d_ref[...], w1b,
                 preferred_element_type=jnp.float32) + b1_ref[...]
    t_ref[...] = hh[:, None, :] + c8[None, :, :]


def _tc_tbuild(h, e2d, w1, b1):
    return pl.pallas_call(
        _tbuild_body,
        grid=(N // NODE_B,),
        in_specs=[
            pl.BlockSpec((NODE_B, D), lambda i: (i, 0)),
            pl.BlockSpec((NUM_ETYPES, 16), lambda i: (0, 0)),
            pl.BlockSpec((D + 16, D), lambda i: (0, 0)),
            pl.BlockSpec((1, D), lambda i: (0, 0)),
        ],
        out_specs=pl.BlockSpec((NODE_B, NUM_ETYPES, D), lambda i: (i, 0, 0)),
        out_shape=jax.ShapeDtypeStruct((N, NUM_ETYPES, D), jnp.float32),
    )(h, e2d, w1, b1)


def _make_sc_edge(lo):
    hi = lo + M

    @functools.partial(
        pl.kernel,
        out_type=(
            jax.ShapeDtypeStruct((NC * H, D), jnp.float32),
            jax.ShapeDtypeStruct((NW * H,), jnp.float32),
        ),
        mesh=_MESH,
        scratch_types=[
            pltpu.VMEM((PER_W + 2 * CHUNK + 16,), jnp.int32),
            pltpu.VMEM((PER_W + 2 * CHUNK + 16,), jnp.int32),
            pltpu.VMEM((NROWS, CHUNK), jnp.int32),
            pltpu.VMEM((D,), jnp.float32),
            pltpu.VMEM((CHUNK, D), jnp.float32),
            pltpu.VMEM((CHUNK, D), jnp.float32),
            pltpu.VMEM((H,), jnp.float32),
            pltpu.VMEM_SHARED((H, D), jnp.float32),
            pltpu.SemaphoreType.DMA,
            pltpu.SemaphoreType.DMA,
        ],
        compiler_params=pltpu.CompilerParams(needs_layout_passes=False),
    )
    def sc_edge(t_hbm, src_hbm, et_hbm, dst_hbm, w3_hbm, zrow_hbm, zden_hbm,
                num_hbm, den_hbm, idx8, stg, didx2, w3v, rows_a,
                rows_b, den_l, sh_num, sem_a, sem_b):
        cid = lax.axis_index("c")
        sid = lax.axis_index("s")
        wid = sid * NC + cid
        base = wid * PER_W

        @pl.when(sid == 0)
        def _():
            pltpu.sync_copy(zrow_hbm, sh_num)

        pltpu.sync_copy(zden_hbm, den_l)
        pltpu.sync_copy(w3_hbm, w3v)
        pltpu.sync_copy(src_hbm.at[pl.ds(base, PER_W)], idx8.at[pl.ds(0, PER_W)])
        pltpu.sync_copy(et_hbm.at[pl.ds(base, PER_W)], stg.at[pl.ds(0, PER_W)])

        def mk_idx(g, carry):
            off = g * 16
            idx8[pl.ds(off, 16)] = idx8[pl.ds(off, 16)] * 8 + stg[pl.ds(off, 16)]
            return carry

        lax.fori_loop(0, PER_W // 16, mk_idx, 0)
        pltpu.sync_copy(dst_hbm.at[pl.ds(base, PER_W)], stg.at[pl.ds(0, PER_W)])

        def compact(g, cnt):
            off = g * 16
            dv = stg[pl.ds(off, 16)]
            iv = idx8[pl.ds(off, 16)]
            inr = (dv >= lo) & (dv < hi)
            plsc.store_compressed(idx8.at[pl.ds(cnt, 16)], iv, mask=inr)
            plsc.store_compressed(stg.at[pl.ds(cnt, 16)], dv - lo, mask=inr)
            return cnt + jnp.sum(inr.astype(jnp.int32))

        n_w = lax.fori_loop(0, PER_W // 16, compact, jnp.int32(0))
        for jj in range(2 * CHUNK // 16):
            idx8[pl.ds(n_w + jj * 16, 16)] = jnp.zeros((16,), jnp.int32)
            stg[pl.ds(n_w + jj * 16, 16)] = jnp.full((16,), DUMP, jnp.int32)
        nci = (n_w + CHUNK - 1) // CHUNK
        nci = nci + 1 - (nci & 1)

        def mk_rows(g2, carry):
            didx2[g2 // (CHUNK // 16),
                  pl.ds((g2 % (CHUNK // 16)) * 16, 16)] = stg[pl.ds(g2 * 16, 16)]
            return carry

        lax.fori_loop(0, nci * (CHUNK // 16), mk_rows, 0)
        plsc.subcore_barrier()

        lane0 = lax.iota(jnp.int32, 16) == 0

        def start(ci, buf, sem):
            pltpu.async_copy(
                t_hbm.at[idx8.at[pl.ds(ci * CHUNK, CHUNK)]], buf, sem)

        def finish(ci, buf, sem):
            pltpu.make_async_copy(
                t_hbm.at[idx8.at[pl.ds(ci * CHUNK, CHUNK)]], buf, sem).wait()

            @plsc.parallel_loop(0, CHUNK, step=1, unroll=4)
            def edge_body(e):
                sigs = []
                acc = None
                for j in range(D // 16):
                    t = buf[e, pl.ds(j * 16, 16)]
                    sj = 1.0 / (1.0 + jnp.exp(-t))
                    sigs.append(sj)
                    term = sj * w3v[pl.ds(j * 16, 16)]
                    acc = term if acc is None else acc + term
                s = jnp.sum(acc)
                exv = jnp.exp(jnp.full((16,), s, jnp.float32))
                for j in range(D // 16):
                    buf[e, pl.ds(j * 16, 16)] = sigs[j] * exv
                dmsplat = plsc.load_gather(
                    stg, [jnp.full((16,), ci * CHUNK + e, jnp.int32)])
                plsc.addupdate_scatter(den_l, [dmsplat], exv, mask=lane0)
            pltpu.sync_copy(buf, sh_num.at[didx2.at[ci]], add=True)

        start(0, rows_a, sem_a)

        def body(k, carry):
            start(2 * k + 1, rows_b, sem_b)
            finish(2 * k, rows_a, sem_a)
            start(2 * k + 2, rows_a, sem_a)
            finish(2 * k + 1, rows_b, sem_b)
            return carry

        lax.fori_loop(0, (nci - 1) // 2, body, 0)
        finish(nci - 1, rows_a, sem_a)
        plsc.subcore_barrier()
        pltpu.sync_copy(den_l, den_hbm.at[pl.ds(wid * H, H)])

        @pl.when(sid == 0)
        def _():
            pltpu.sync_copy(sh_num, num_hbm.at[pl.ds(cid * H, H)])

    return sc_edge


_SC_EDGE = (_make_sc_edge(0), _make_sc_edge(M))


def _update_body(num_ref, den_ref, h_ref, w2_ref, b2_ref, out_ref,
                 *, alpha, beta, lo):
    ones = jnp.ones((NW, 1), jnp.float32)
    den = lax.dot_general(den_ref[...], ones, (((0,), (0,)), ((), ())),
                          preferred_element_type=jnp.float32)
    num = num_ref[pl.ds(0, H), :] + num_ref[pl.ds(H, H), :]
    agg = (num / jnp.maximum(den, 1e-30))[:M, :]
    z = alpha * agg + beta * h_ref[pl.ds(lo, M), :]
    z = jax.nn.sigmoid(jnp.dot(z, w2_ref[...],
                               preferred_element_type=jnp.float32) + b2_ref[...])
    if lo == 0:
        out_ref[pl.ds(0, M), :] = z
        out_ref[pl.ds(M, N - M), :] = h_ref[pl.ds(M, N - M), :]
    else:
        out_ref[pl.ds(0, lo), :] = h_ref[pl.ds(0, lo), :]
        out_ref[pl.ds(lo, M), :] = z


def _tc_update(num, den, h, w2, b2, alpha, beta, lo):
    body = functools.partial(_update_body, alpha=alpha, beta=beta, lo=lo)
    return pl.pallas_call(
        body,
        out_shape=jax.ShapeDtypeStruct((N, D), jnp.float32),
    )(num, den, h, w2, b2)


def kernel(features, edge_index, edge_type, w1_wkr, b1_wkr, w1_tsk, b1_tsk,
           w2_wkr, b2_wkr, w2_tsk, b2_tsk, w3_wkr, b3_wkr, w3_tsk, b3_tsk,
           e_wkr, e_tsk):
    h0 = features[:, 0, :]
    src = edge_index[0].astype(jnp.int32)
    dst = edge_index[1].astype(jnp.int32)
    et = edge_type.astype(jnp.int32)
    zrow = jnp.zeros((H, D), jnp.float32)
    zden = jnp.zeros((H,), jnp.float32)

    def stage(h, e_tab, w1, b1, w3, idx):
        t_tab = _tc_tbuild(h, e_tab[:, 0, :], w1, b1).reshape(N * NUM_ETYPES, D)
        numf, denf = _SC_EDGE[idx](t_tab, src, et, dst, w3[:D, 0], zrow, zden)
        return numf, denf.reshape(NW, H)

    num, den = stage(h0, e_wkr, w1_wkr, b1_wkr, w3_wkr, 0)
    h1 = _tc_update(num, den, h0, w2_wkr, b2_wkr, 1.0, 0.5, 0)
    num, den = stage(h1, e_tsk, w1_tsk, b1_tsk, w3_tsk, 1)
    h2 = _tc_update(num, den, h1, w2_tsk, b2_tsk, 0.7, 0.3, NUM_WKR)
    return h2[:, None, :]
